# Initial kernel scaffold; baseline (speedup 1.0000x reference)
import jax
import jax.numpy as jnp
from jax.experimental import pallas as pl


def kernel(weights, starts, ends):
    raise NotImplementedError("write your pallas kernel here")


# speedup vs baseline: 12.4837x; 12.4837x over previous
import functools

import jax
import jax.numpy as jnp
from jax.experimental import pallas as pl
from jax.experimental.pallas import tpu as pltpu

_BLK = 1024
_S = 128
_SPLIT = 0.55


def _tc_body(w_ref, st_ref, en_ref, out_ref):
    w = w_ref[...]
    steps = (st_ref[...] + en_ref[...]) * 0.5

    row = jax.lax.broadcasted_iota(jnp.int32, (_S, _S), 0)
    col = jax.lax.broadcasted_iota(jnp.int32, (_S, _S), 1)
    tri = (row <= col).astype(jnp.float32)
    cw = jax.lax.dot_general(
        w, tri, (((1,), (0,)), ((), ())),
        preferred_element_type=jnp.float32,
        precision=jax.lax.Precision.HIGHEST,
    )

    cnt = jnp.sum((cw < _SPLIT).astype(jnp.int32), axis=1, keepdims=True)
    idx = jnp.minimum(cnt, _S - 1)
    lane = jax.lax.broadcasted_iota(jnp.int32, (_BLK, _S), 1)
    sel = jnp.where(lane == idx, steps, 0.0)
    out_ref[...] = jnp.sum(sel, axis=1, keepdims=True)


def kernel(weights, starts, ends):
    B, S = weights.shape[0], weights.shape[1]
    w2 = weights[..., 0]
    st2 = starts[..., 0]
    en2 = ends[..., 0]
    grid = (B // _BLK,)
    out = pl.pallas_call(
        _tc_body,
        grid=grid,
        in_specs=[
            pl.BlockSpec((_BLK, S), lambda i: (i, 0)),
            pl.BlockSpec((_BLK, S), lambda i: (i, 0)),
            pl.BlockSpec((_BLK, S), lambda i: (i, 0)),
        ],
        out_specs=pl.BlockSpec((_BLK, 1), lambda i: (i, 0)),
        out_shape=jax.ShapeDtypeStruct((B, 1), jnp.float32),
    )(w2, st2, en2)
    return out


# speedup vs baseline: 56.6861x; 4.5408x over previous
import functools

import jax
import jax.numpy as jnp
from jax import lax
from jax.experimental import pallas as pl
from jax.experimental.pallas import tpu as pltpu
from jax.experimental.pallas import tpu_sc as plsc

_B = 131072
_S = 128
_SPLIT = 0.55
_L = 16
_NW = 32
_RPW = _B // _NW
_CH = 16
_GROUPS = _RPW // _L
_NSTREAM = _RPW // 128
_GPS = 128 // _L


def _scan_chunk(src_ref, row, acc, cnt):
    for t in range(_L):
        col = jnp.full((_L,), t, jnp.int32)
        w = plsc.load_gather(src_ref, [row, col])
        acc = acc + w
        cnt = cnt + jnp.where(acc < _SPLIT, 1, 0)
    return acc, cnt


def _sc_body(w_hbm, st_hbm, en_hbm, out_hbm, w_v, w2_v, idx_v, s_v, e_v, o_v, sem):
    wid = lax.axis_index("s") * 2 + lax.axis_index("c")
    base = wid * _RPW
    pltpu.sync_copy(w_hbm.at[pl.ds(base, _RPW), pl.ds(0, _CH)], w_v)

    iota = lax.broadcasted_iota(jnp.int32, (_L,), 0)

    def stream_body(j, _):
        def group_body(gg, _):
            g = j * _GPS + gg
            row = g * _L + iota
            acc = jnp.zeros((_L,), jnp.float32)
            cnt = jnp.zeros((_L,), jnp.int32)
            acc, cnt = _scan_chunk(w_v, row, acc, cnt)

            def fb_cond(carry):
                c, _a, _n, m = carry
                return jnp.logical_and(c < _S // _L, m < _SPLIT)

            def fb_body(carry):
                c, a, n, _m = carry
                pltpu.sync_copy(
                    w_hbm.at[pl.ds(base + g * _L, _L), pl.ds(c * _L, _L)],
                    w2_v)
                a, n = _scan_chunk(w2_v, iota, a, n)
                return (c + 1, a, n, jnp.min(a))

            carry_out = lax.while_loop(
                fb_cond, fb_body,
                (jnp.int32(1), acc, cnt, jnp.min(acc)))
            cnt = carry_out[2]

            idx = jnp.minimum(cnt, _S - 1)
            idx_v[pl.ds(g * _L, _L)] = (base + row) * _S + idx
            return None

        lax.fori_loop(0, _GPS, group_body, None)
        sl = pl.ds(j * 128, 128)
        pltpu.async_copy(st_hbm.at[idx_v.at[sl]], s_v.at[sl], sem)
        pltpu.async_copy(en_hbm.at[idx_v.at[sl]], e_v.at[sl], sem)
        return None

    lax.fori_loop(0, _NSTREAM, stream_body, None)

    def drain_body(j, _):
        sl = pl.ds(j * 128, 128)
        pltpu.make_async_copy(st_hbm.at[idx_v.at[sl]], s_v.at[sl], sem).wait()
        pltpu.make_async_copy(en_hbm.at[idx_v.at[sl]], e_v.at[sl], sem).wait()
        return None

    lax.fori_loop(0, _NSTREAM, drain_body, None)

    def avg_body(k, _):
        sl = pl.ds(k * _L, _L)
        o_v[sl] = (s_v[sl] + e_v[sl]) * 0.5
        return None

    lax.fori_loop(0, _RPW // _L, avg_body, None)
    pltpu.sync_copy(o_v, out_hbm.at[pl.ds(base, _RPW)])


@jax.jit
def _sc_call(w2, st_flat, en_flat):
    mesh = plsc.VectorSubcoreMesh(core_axis_name="c", subcore_axis_name="s")
    f = pl.kernel(
        _sc_body,
        out_type=jax.ShapeDtypeStruct((_B,), jnp.float32),
        mesh=mesh,
        scratch_types=[
            pltpu.VMEM((_RPW, _CH), jnp.float32),
            pltpu.VMEM((_L, _L), jnp.float32),
            pltpu.VMEM((_RPW,), jnp.int32),
            pltpu.VMEM((_RPW,), jnp.float32),
            pltpu.VMEM((_RPW,), jnp.float32),
            pltpu.VMEM((_RPW,), jnp.float32),
            pltpu.SemaphoreType.DMA,
        ],
        compiler_params=pltpu.CompilerParams(
            use_tc_tiling_on_sc=False, needs_layout_passes=False),
    )
    return f(w2, st_flat, en_flat)


def kernel(weights, starts, ends):
    B = weights.shape[0]
    w2 = weights[..., 0]
    st_flat = starts.reshape(-1)
    en_flat = ends.reshape(-1)
    out = _sc_call(w2, st_flat, en_flat)
    return out.reshape(B, 1)
